# Initial kernel scaffold; baseline (speedup 1.0000x reference)
import functools
import numpy as np
import jax
import jax.numpy as jnp
from jax.experimental import pallas as pl
from jax.experimental.pallas import tpu as pltpu

N = 10000
E = 160000
C = 128
NB = 8
RMAX = 5.0
NL = 2
NELEM = 119
AVG_NEIGH = 16.0

PATH = (0, 1, 1, 1, 2, 2, 2, 2, 2)
SCALE = 1.0 / np.sqrt(C)

BE = 640
BN = 500


def _silu(x):
    return x * (1.0 / (1.0 + jnp.exp(-x)))


def _pll(h2, W):
    outs = [jnp.dot(h2[:, m * C:(m + 1) * C], W[PATH[m]],
                    preferred_element_type=jnp.float32) for m in range(9)]
    return jnp.concatenate(outs, axis=1) * SCALE


def _edge_body(ef_ref, y_ref, x_ref, wr1, br1, wr2, br2, wr3, br3, cgf, msg_ref):
    ef = ef_ref[...]
    t = _silu(jnp.dot(ef, wr1[...], preferred_element_type=jnp.float32) + br1[...])
    t = _silu(jnp.dot(t, wr2[...], preferred_element_type=jnp.float32) + br2[...])
    w = jnp.dot(t, wr3[...], preferred_element_type=jnp.float32) + br3[...]
    g = jnp.dot(y_ref[...], cgf[...], preferred_element_type=jnp.float32)
    x = x_ref[...]
    inv = 1.0 / AVG_NEIGH
    for k in range(9):
        acc = x[:, 0:C] * g[:, k:k + 1]
        for i in range(1, 9):
            acc = acc + x[:, i * C:(i + 1) * C] * g[:, i * 9 + k:i * 9 + k + 1]
        p = PATH[k]
        msg_ref[:, k * C:(k + 1) * C] = acc * w[:, p * C:(p + 1) * C] * inv


def _edge_stage(ef, y, x2, wr1, br1, wr2, br2, wr3, br3, cgf):
    grid = (E // BE,)
    return pl.pallas_call(
        _edge_body,
        grid=grid,
        in_specs=[
            pl.BlockSpec((BE, NB), lambda i: (i, 0)),
            pl.BlockSpec((BE, 9), lambda i: (i, 0)),
            pl.BlockSpec((BE, 9 * C), lambda i: (i, 0)),
            pl.BlockSpec((NB, C), lambda i: (0, 0)),
            pl.BlockSpec((1, C), lambda i: (0, 0)),
            pl.BlockSpec((C, C), lambda i: (0, 0)),
            pl.BlockSpec((1, C), lambda i: (0, 0)),
            pl.BlockSpec((C, 3 * C), lambda i: (0, 0)),
            pl.BlockSpec((1, 3 * C), lambda i: (0, 0)),
            pl.BlockSpec((9, 81), lambda i: (0, 0)),
        ],
        out_specs=pl.BlockSpec((BE, 9 * C), lambda i: (i, 0)),
        out_shape=jax.ShapeDtypeStruct((E, 9 * C), jnp.float32),
    )(ef, y, x2, wr1, br1, wr2, br2, wr3, br3, cgf)


def _nodeA_body(h_ref, w1_ref, out_ref):
    out_ref[...] = _pll(h_ref[...], w1_ref[...])


def _nodeA(h2, W1l):
    return pl.pallas_call(
        _nodeA_body,
        grid=(N // BN,),
        in_specs=[
            pl.BlockSpec((BN, 9 * C), lambda i: (i, 0)),
            pl.BlockSpec((3, C, C), lambda i: (0, 0, 0)),
        ],
        out_specs=pl.BlockSpec((BN, 9 * C), lambda i: (i, 0)),
        out_shape=jax.ShapeDtypeStruct((N, 9 * C), jnp.float32),
    )(h2, W1l)


def _nodeB_body(agg_ref, sc_ref, w2_ref, ac_ref, wp_ref, out_ref):
    m = _pll(agg_ref[...], w2_ref[...])
    s = m[:, 0:C]
    ac = ac_ref[...]
    fac = ac[0:1, :] + ac[1:2, :] * s + ac[2:3, :] * (s * s)
    body = jnp.concatenate([m[:, j * C:(j + 1) * C] * fac for j in range(9)], axis=1)
    out_ref[...] = _pll(body, wp_ref[...]) + sc_ref[...]


def _nodeB(agg2, sc2, W2l, acl, Wpl):
    return pl.pallas_call(
        _nodeB_body,
        grid=(N // BN,),
        in_specs=[
            pl.BlockSpec((BN, 9 * C), lambda i: (i, 0)),
            pl.BlockSpec((BN, 9 * C), lambda i: (i, 0)),
            pl.BlockSpec((3, C, C), lambda i: (0, 0, 0)),
            pl.BlockSpec((3, C), lambda i: (0, 0)),
            pl.BlockSpec((3, C, C), lambda i: (0, 0, 0)),
        ],
        out_specs=pl.BlockSpec((BN, 9 * C), lambda i: (i, 0)),
        out_shape=jax.ShapeDtypeStruct((N, 9 * C), jnp.float32),
    )(agg2, sc2, W2l, acl, Wpl)


def kernel(Z, bond_dist, bond_diff, edge_index, embed, W1, Wr1, br1, Wr2, br2,
           Wr3, br3, CGw, W2, acoef, Wp):
    r = bond_dist
    edge_dir = bond_diff / (r[:, None] + 1e-8)
    x, y, z = edge_dir[:, 0], edge_dir[:, 1], edge_dir[:, 2]
    s3, s15, s5 = np.sqrt(3.0), np.sqrt(15.0), np.sqrt(5.0)
    Y = jnp.stack([jnp.ones_like(x), s3 * x, s3 * y, s3 * z,
                   s15 * x * y, s15 * y * z, (s5 / 2.0) * (3.0 * z * z - 1.0),
                   s15 * x * z, (s15 / 2.0) * (x * x - y * y)], axis=-1)
    nb = jnp.arange(1, NB + 1, dtype=jnp.float32)
    rr = r[:, None] + 1e-8
    bes = np.sqrt(2.0 / RMAX) * jnp.sin(nb * np.pi * rr / RMAX) / rr
    cut = 0.5 * (jnp.cos(np.pi * r / RMAX) + 1.0) * (r < RMAX).astype(jnp.float32)
    ef = bes * cut[:, None]

    src = edge_index[:, 0]
    dst = edge_index[:, 1]

    node_scalar = embed[Z]
    h2 = jnp.zeros((N, 9 * C), jnp.float32).at[:, 0:C].set(node_scalar)

    br1_2 = br1.reshape(NL, 1, C)
    br2_2 = br2.reshape(NL, 1, C)
    br3_2 = br3.reshape(NL, 1, 3 * C)

    for layer in range(NL):
        cgf = jnp.transpose(CGw[layer], (2, 1, 0)).reshape(9, 81)
        h_up = _nodeA(h2, W1[layer])
        x2 = jnp.take(h_up, src, axis=0)
        msg = _edge_stage(ef, Y, x2, Wr1[layer], br1_2[layer], Wr2[layer],
                          br2_2[layer], Wr3[layer], br3_2[layer], cgf)
        agg = jax.ops.segment_sum(msg, dst, num_segments=N)
        h2 = _nodeB(agg, h2, W2[layer], acoef[layer], Wp[layer])

    return h2.reshape(N, 9, C).transpose(0, 2, 1).reshape(N, 9 * C)


# speedup vs baseline: 4.5672x; 4.5672x over previous
import functools
import numpy as np
import jax
import jax.numpy as jnp
from jax.experimental import pallas as pl
from jax.experimental.pallas import tpu as pltpu

N = 10000
E = 160000
C = 128
NB = 8
RMAX = 5.0
NL = 2
NELEM = 119
AVG_NEIGH = 16.0

PATH = (0, 1, 1, 1, 2, 2, 2, 2, 2)
SCALE = 1.0 / np.sqrt(C)

BE = 640
BN = 1000


def _silu(x):
    return x * (1.0 / (1.0 + jnp.exp(-x)))


def _pll(h2, W):
    outs = [jnp.dot(h2[:, m * C:(m + 1) * C], W[PATH[m]],
                    preferred_element_type=jnp.float32) for m in range(9)]
    return jnp.concatenate(outs, axis=1) * SCALE


def _edge_body(ef_ref, y_ref, x_ref, wr1, br1, wr2, br2, wr3, br3, cgf, msg_ref):
    ef = ef_ref[...]
    t = _silu(jnp.dot(ef, wr1[...], preferred_element_type=jnp.float32) + br1[...])
    t = _silu(jnp.dot(t, wr2[...], preferred_element_type=jnp.float32) + br2[...])
    w = jnp.dot(t, wr3[...], preferred_element_type=jnp.float32) + br3[...]
    g = jnp.dot(y_ref[...], cgf[...], preferred_element_type=jnp.float32)
    x = x_ref[...]
    inv = 1.0 / AVG_NEIGH
    for k in range(9):
        acc = x[:, 0:C] * g[:, k:k + 1]
        for i in range(1, 9):
            acc = acc + x[:, i * C:(i + 1) * C] * g[:, i * 9 + k:i * 9 + k + 1]
        p = PATH[k]
        msg_ref[:, k * C:(k + 1) * C] = acc * w[:, p * C:(p + 1) * C] * inv


def _edge_stage(ef, y, x2, wr1, br1, wr2, br2, wr3, br3, cgf):
    grid = (E // BE,)
    return pl.pallas_call(
        _edge_body,
        grid=grid,
        in_specs=[
            pl.BlockSpec((BE, NB), lambda i: (i, 0)),
            pl.BlockSpec((BE, 9), lambda i: (i, 0)),
            pl.BlockSpec((BE, 9 * C), lambda i: (i, 0)),
            pl.BlockSpec((NB, C), lambda i: (0, 0)),
            pl.BlockSpec((1, C), lambda i: (0, 0)),
            pl.BlockSpec((C, C), lambda i: (0, 0)),
            pl.BlockSpec((1, C), lambda i: (0, 0)),
            pl.BlockSpec((C, 3 * C), lambda i: (0, 0)),
            pl.BlockSpec((1, 3 * C), lambda i: (0, 0)),
            pl.BlockSpec((9, 81), lambda i: (0, 0)),
        ],
        out_specs=pl.BlockSpec((BE, 9 * C), lambda i: (i, 0)),
        out_shape=jax.ShapeDtypeStruct((E, 9 * C), jnp.float32),
    )(ef, y, x2, wr1, br1, wr2, br2, wr3, br3, cgf)


def _nodeA_body(h_ref, w1_ref, out_ref):
    out_ref[...] = _pll(h_ref[...], w1_ref[...])


def _nodeA(h2, W1l):
    return pl.pallas_call(
        _nodeA_body,
        grid=(N // BN,),
        in_specs=[
            pl.BlockSpec((BN, 9 * C), lambda i: (i, 0)),
            pl.BlockSpec((3, C, C), lambda i: (0, 0, 0)),
        ],
        out_specs=pl.BlockSpec((BN, 9 * C), lambda i: (i, 0)),
        out_shape=jax.ShapeDtypeStruct((N, 9 * C), jnp.float32),
    )(h2, W1l)


def _nodeB_body(agg_ref, sc_ref, w2_ref, ac_ref, wp_ref, out_ref):
    m = _pll(agg_ref[...], w2_ref[...])
    s = m[:, 0:C]
    ac = ac_ref[...]
    fac = ac[0:1, :] + ac[1:2, :] * s + ac[2:3, :] * (s * s)
    body = jnp.concatenate([m[:, j * C:(j + 1) * C] * fac for j in range(9)], axis=1)
    out_ref[...] = _pll(body, wp_ref[...]) + sc_ref[...]


def _nodeB(agg2, sc2, W2l, acl, Wpl):
    return pl.pallas_call(
        _nodeB_body,
        grid=(N // BN,),
        in_specs=[
            pl.BlockSpec((BN, 9 * C), lambda i: (i, 0)),
            pl.BlockSpec((BN, 9 * C), lambda i: (i, 0)),
            pl.BlockSpec((3, C, C), lambda i: (0, 0, 0)),
            pl.BlockSpec((3, C), lambda i: (0, 0)),
            pl.BlockSpec((3, C, C), lambda i: (0, 0, 0)),
        ],
        out_specs=pl.BlockSpec((BN, 9 * C), lambda i: (i, 0)),
        out_shape=jax.ShapeDtypeStruct((N, 9 * C), jnp.float32),
    )(agg2, sc2, W2l, acl, Wpl)


def kernel(Z, bond_dist, bond_diff, edge_index, embed, W1, Wr1, br1, Wr2, br2,
           Wr3, br3, CGw, W2, acoef, Wp):
    r = bond_dist
    edge_dir = bond_diff / (r[:, None] + 1e-8)
    x, y, z = edge_dir[:, 0], edge_dir[:, 1], edge_dir[:, 2]
    s3, s15, s5 = np.sqrt(3.0), np.sqrt(15.0), np.sqrt(5.0)
    Y = jnp.stack([jnp.ones_like(x), s3 * x, s3 * y, s3 * z,
                   s15 * x * y, s15 * y * z, (s5 / 2.0) * (3.0 * z * z - 1.0),
                   s15 * x * z, (s15 / 2.0) * (x * x - y * y)], axis=-1)
    nb = jnp.arange(1, NB + 1, dtype=jnp.float32)
    rr = r[:, None] + 1e-8
    bes = np.sqrt(2.0 / RMAX) * jnp.sin(nb * np.pi * rr / RMAX) / rr
    cut = 0.5 * (jnp.cos(np.pi * r / RMAX) + 1.0) * (r < RMAX).astype(jnp.float32)
    ef = bes * cut[:, None]

    src = edge_index[:, 0]
    dst = edge_index[:, 1]

    node_scalar = embed[Z]
    h2 = jnp.zeros((N, 9 * C), jnp.float32).at[:, 0:C].set(node_scalar)

    br1_2 = br1.reshape(NL, 1, C)
    br2_2 = br2.reshape(NL, 1, C)
    perm = np.arange(3 * C).reshape(3, C)
    perm = (perm % C) * 3 + (perm // C)
    perm = perm.reshape(-1)
    Wr3_p = Wr3[:, :, perm]
    br3_2 = br3[:, perm].reshape(NL, 1, 3 * C)

    for layer in range(NL):
        cgf = jnp.transpose(CGw[layer], (2, 1, 0)).reshape(9, 81)
        h_up = _nodeA(h2, W1[layer])
        x2 = jnp.take(h_up, src, axis=0)
        msg = _edge_stage(ef, Y, x2, Wr1[layer], br1_2[layer], Wr2[layer],
                          br2_2[layer], Wr3_p[layer], br3_2[layer], cgf)
        agg = jax.ops.segment_sum(msg, dst, num_segments=N)
        h2 = _nodeB(agg, h2, W2[layer], acoef[layer], Wp[layer])

    return h2.reshape(N, 9, C).transpose(0, 2, 1).reshape(N, 9 * C)


# speedup vs baseline: 5.4477x; 1.1928x over previous
import functools
import numpy as np
import jax
import jax.numpy as jnp
from jax import lax
from jax.experimental import pallas as pl
from jax.experimental.pallas import tpu as pltpu
from jax.experimental.pallas import tpu_sc as plsc

N = 10000
E = 160000
C = 128
NB = 8
RMAX = 5.0
NL = 2
NELEM = 119
AVG_NEIGH = 16.0

PATH = (0, 1, 1, 1, 2, 2, 2, 2, 2)
SCALE = 1.0 / np.sqrt(C)

BE = 640
BN = 1000


def _silu(x):
    return x * (1.0 / (1.0 + jnp.exp(-x)))


def _pll(h2, W):
    outs = [jnp.dot(h2[:, m * C:(m + 1) * C], W[PATH[m]],
                    preferred_element_type=jnp.float32) for m in range(9)]
    return jnp.concatenate(outs, axis=1) * SCALE


def _edge_body(ef_ref, y_ref, x_ref, wr1, br1, wr2, br2, wr3, br3, cgf, msg_ref):
    ef = ef_ref[...]
    t = _silu(jnp.dot(ef, wr1[...], preferred_element_type=jnp.float32) + br1[...])
    t = _silu(jnp.dot(t, wr2[...], preferred_element_type=jnp.float32) + br2[...])
    w = jnp.dot(t, wr3[...], preferred_element_type=jnp.float32) + br3[...]
    g = jnp.dot(y_ref[...], cgf[...], preferred_element_type=jnp.float32)
    x = x_ref[...]
    inv = 1.0 / AVG_NEIGH
    for k in range(9):
        acc = x[:, 0:C] * g[:, k:k + 1]
        for i in range(1, 9):
            acc = acc + x[:, i * C:(i + 1) * C] * g[:, i * 9 + k:i * 9 + k + 1]
        p = PATH[k]
        msg_ref[:, k * C:(k + 1) * C] = acc * w[:, p * C:(p + 1) * C] * inv


def _edge_stage(ef, y, x2, wr1, br1, wr2, br2, wr3, br3, cgf):
    grid = (E // BE,)
    return pl.pallas_call(
        _edge_body,
        grid=grid,
        in_specs=[
            pl.BlockSpec((BE, NB), lambda i: (i, 0)),
            pl.BlockSpec((BE, 9), lambda i: (i, 0)),
            pl.BlockSpec((BE, 9 * C), lambda i: (i, 0)),
            pl.BlockSpec((NB, C), lambda i: (0, 0)),
            pl.BlockSpec((1, C), lambda i: (0, 0)),
            pl.BlockSpec((C, C), lambda i: (0, 0)),
            pl.BlockSpec((1, C), lambda i: (0, 0)),
            pl.BlockSpec((C, 3 * C), lambda i: (0, 0)),
            pl.BlockSpec((1, 3 * C), lambda i: (0, 0)),
            pl.BlockSpec((9, 81), lambda i: (0, 0)),
        ],
        out_specs=pl.BlockSpec((BE, 9 * C), lambda i: (i, 0)),
        out_shape=jax.ShapeDtypeStruct((E, 9 * C), jnp.float32),
    )(ef, y, x2, wr1, br1, wr2, br2, wr3, br3, cgf)


NW = 32
EPW = E // NW
KG = 40
NBATCH = EPW // KG

_SC_MESH = plsc.VectorSubcoreMesh(core_axis_name="c", subcore_axis_name="s")


@functools.partial(
    pl.kernel,
    mesh=_SC_MESH,
    out_type=jax.ShapeDtypeStruct((E, 9 * C), jnp.float32),
    scratch_types=[
        pltpu.VMEM((EPW,), jnp.int32),
        pltpu.VMEM((KG, 9 * C), jnp.float32),
        pltpu.VMEM((KG, 9 * C), jnp.float32),
        pltpu.SemaphoreType.DMA,
        pltpu.SemaphoreType.DMA,
    ],
)
def _sc_gather(tab_hbm, idx_hbm, out_hbm, idx_v, buf0, buf1, sem0, sem1):
    wid = lax.axis_index("s") * 2 + lax.axis_index("c")
    base = wid * EPW
    pltpu.sync_copy(idx_hbm.at[pl.ds(base, EPW)], idx_v)

    def gath(j, buf, sem):
        pltpu.async_copy(tab_hbm.at[idx_v.at[pl.ds(j * KG, KG)]], buf, sem)

    def wait_g(buf, sem):
        pltpu.make_async_copy(tab_hbm.at[idx_v.at[pl.ds(0, KG)]], buf, sem).wait()

    def wback(j, buf):
        pltpu.sync_copy(buf, out_hbm.at[pl.ds(base + j * KG, KG)])

    gath(0, buf0, sem0)

    def body(j2, _):
        j = j2 * 2
        wait_g(buf0, sem0)
        gath(j + 1, buf1, sem1)
        wback(j, buf0)
        wait_g(buf1, sem1)

        @pl.when(j + 2 < NBATCH)
        def _():
            gath(j + 2, buf0, sem0)
        wback(j + 1, buf1)
        return 0

    lax.fori_loop(0, NBATCH // 2, body, 0)
    if NBATCH % 2 == 1:
        wait_g(buf0, sem0)
        wback(NBATCH - 1, buf0)


def _nodeA_body(h_ref, w1_ref, out_ref):
    out_ref[...] = _pll(h_ref[...], w1_ref[...])


def _nodeA(h2, W1l):
    return pl.pallas_call(
        _nodeA_body,
        grid=(N // BN,),
        in_specs=[
            pl.BlockSpec((BN, 9 * C), lambda i: (i, 0)),
            pl.BlockSpec((3, C, C), lambda i: (0, 0, 0)),
        ],
        out_specs=pl.BlockSpec((BN, 9 * C), lambda i: (i, 0)),
        out_shape=jax.ShapeDtypeStruct((N, 9 * C), jnp.float32),
    )(h2, W1l)


def _nodeB_body(agg_ref, sc_ref, w2_ref, ac_ref, wp_ref, out_ref):
    m = _pll(agg_ref[...], w2_ref[...])
    s = m[:, 0:C]
    ac = ac_ref[...]
    fac = ac[0:1, :] + ac[1:2, :] * s + ac[2:3, :] * (s * s)
    body = jnp.concatenate([m[:, j * C:(j + 1) * C] * fac for j in range(9)], axis=1)
    out_ref[...] = _pll(body, wp_ref[...]) + sc_ref[...]


def _nodeB(agg2, sc2, W2l, acl, Wpl):
    return pl.pallas_call(
        _nodeB_body,
        grid=(N // BN,),
        in_specs=[
            pl.BlockSpec((BN, 9 * C), lambda i: (i, 0)),
            pl.BlockSpec((BN, 9 * C), lambda i: (i, 0)),
            pl.BlockSpec((3, C, C), lambda i: (0, 0, 0)),
            pl.BlockSpec((3, C), lambda i: (0, 0)),
            pl.BlockSpec((3, C, C), lambda i: (0, 0, 0)),
        ],
        out_specs=pl.BlockSpec((BN, 9 * C), lambda i: (i, 0)),
        out_shape=jax.ShapeDtypeStruct((N, 9 * C), jnp.float32),
    )(agg2, sc2, W2l, acl, Wpl)


def kernel(Z, bond_dist, bond_diff, edge_index, embed, W1, Wr1, br1, Wr2, br2,
           Wr3, br3, CGw, W2, acoef, Wp):
    r = bond_dist
    edge_dir = bond_diff / (r[:, None] + 1e-8)
    x, y, z = edge_dir[:, 0], edge_dir[:, 1], edge_dir[:, 2]
    s3, s15, s5 = np.sqrt(3.0), np.sqrt(15.0), np.sqrt(5.0)
    Y = jnp.stack([jnp.ones_like(x), s3 * x, s3 * y, s3 * z,
                   s15 * x * y, s15 * y * z, (s5 / 2.0) * (3.0 * z * z - 1.0),
                   s15 * x * z, (s15 / 2.0) * (x * x - y * y)], axis=-1)
    nb = jnp.arange(1, NB + 1, dtype=jnp.float32)
    rr = r[:, None] + 1e-8
    bes = np.sqrt(2.0 / RMAX) * jnp.sin(nb * np.pi * rr / RMAX) / rr
    cut = 0.5 * (jnp.cos(np.pi * r / RMAX) + 1.0) * (r < RMAX).astype(jnp.float32)
    ef = bes * cut[:, None]

    src = edge_index[:, 0]
    dst = edge_index[:, 1]

    node_scalar = embed[Z]
    h2 = jnp.zeros((N, 9 * C), jnp.float32).at[:, 0:C].set(node_scalar)

    br1_2 = br1.reshape(NL, 1, C)
    br2_2 = br2.reshape(NL, 1, C)
    perm = np.arange(3 * C).reshape(3, C)
    perm = (perm % C) * 3 + (perm // C)
    perm = perm.reshape(-1)
    Wr3_p = Wr3[:, :, perm]
    br3_2 = br3[:, perm].reshape(NL, 1, 3 * C)

    for layer in range(NL):
        cgf = jnp.transpose(CGw[layer], (2, 1, 0)).reshape(9, 81)
        h_up = _nodeA(h2, W1[layer])
        x2 = _sc_gather(h_up, src)
        msg = _edge_stage(ef, Y, x2, Wr1[layer], br1_2[layer], Wr2[layer],
                          br2_2[layer], Wr3_p[layer], br3_2[layer], cgf)
        agg = jax.ops.segment_sum(msg, dst, num_segments=N)
        h2 = _nodeB(agg, h2, W2[layer], acoef[layer], Wp[layer])

    return h2.reshape(N, 9, C).transpose(0, 2, 1).reshape(N, 9 * C)


# speedup vs baseline: 5.6262x; 1.0328x over previous
import functools
import numpy as np
import jax
import jax.numpy as jnp
from jax import lax
from jax.experimental import pallas as pl
from jax.experimental.pallas import tpu as pltpu
from jax.experimental.pallas import tpu_sc as plsc

N = 10000
E = 160000
C = 128
NB = 8
RMAX = 5.0
NL = 2
NELEM = 119
AVG_NEIGH = 16.0

PATH = (0, 1, 1, 1, 2, 2, 2, 2, 2)
SCALE = 1.0 / np.sqrt(C)

BE = 640
BN = 1000


def _silu(x):
    return x * (1.0 / (1.0 + jnp.exp(-x)))


def _pll(h2, W):
    outs = [jnp.dot(h2[:, m * C:(m + 1) * C], W[PATH[m]],
                    preferred_element_type=jnp.float32) for m in range(9)]
    return jnp.concatenate(outs, axis=1) * SCALE


def _edge_body(ef_ref, y_ref, x_ref, wr1, br1, wr2, br2, wr3, br3, cgf, msg_ref):
    ef = ef_ref[...]
    t = _silu(jnp.dot(ef, wr1[...], preferred_element_type=jnp.float32) + br1[...])
    t = _silu(jnp.dot(t, wr2[...], preferred_element_type=jnp.float32) + br2[...])
    w = jnp.dot(t, wr3[...], preferred_element_type=jnp.float32) + br3[...]
    g = jnp.dot(y_ref[...], cgf[...], preferred_element_type=jnp.float32)
    x = x_ref[...]
    inv = 1.0 / AVG_NEIGH
    for k in range(9):
        acc = x[:, 0:C] * g[:, k:k + 1]
        for i in range(1, 9):
            acc = acc + x[:, i * C:(i + 1) * C] * g[:, i * 9 + k:i * 9 + k + 1]
        p = PATH[k]
        msg_ref[:, k * C:(k + 1) * C] = acc * w[:, p * C:(p + 1) * C] * inv


def _edge_stage(ef, y, x2, wr1, br1, wr2, br2, wr3, br3, cgf):
    grid = (E // BE,)
    return pl.pallas_call(
        _edge_body,
        grid=grid,
        in_specs=[
            pl.BlockSpec((BE, NB), lambda i: (i, 0)),
            pl.BlockSpec((BE, 9), lambda i: (i, 0)),
            pl.BlockSpec((BE, 9 * C), lambda i: (i, 0)),
            pl.BlockSpec((NB, C), lambda i: (0, 0)),
            pl.BlockSpec((1, C), lambda i: (0, 0)),
            pl.BlockSpec((C, C), lambda i: (0, 0)),
            pl.BlockSpec((1, C), lambda i: (0, 0)),
            pl.BlockSpec((C, 3 * C), lambda i: (0, 0)),
            pl.BlockSpec((1, 3 * C), lambda i: (0, 0)),
            pl.BlockSpec((9, 81), lambda i: (0, 0)),
        ],
        out_specs=pl.BlockSpec((BE, 9 * C), lambda i: (i, 0)),
        out_shape=jax.ShapeDtypeStruct((E, 9 * C), jnp.float32),
    )(ef, y, x2, wr1, br1, wr2, br2, wr3, br3, cgf)


NW = 32
EPW = E // NW
KG = 40
NBATCH = EPW // KG

@functools.lru_cache(maxsize=None)
def _get_sc_gather():
    mesh = plsc.VectorSubcoreMesh(core_axis_name="c", subcore_axis_name="s")
    return functools.partial(
        pl.kernel,
        mesh=mesh,
        out_type=jax.ShapeDtypeStruct((E, 9 * C), jnp.float32),
        scratch_types=[
            pltpu.VMEM((EPW,), jnp.int32),
            pltpu.VMEM((KG, 9 * C), jnp.float32),
            pltpu.VMEM((KG, 9 * C), jnp.float32),
            pltpu.SemaphoreType.DMA,
            pltpu.SemaphoreType.DMA,
        ],
    )(_sc_gather_body)


def _sc_gather_body(tab_hbm, idx_hbm, out_hbm, idx_v, buf0, buf1, sem0, sem1):
    wid = lax.axis_index("s") * 2 + lax.axis_index("c")
    base = wid * EPW
    pltpu.sync_copy(idx_hbm.at[pl.ds(base, EPW)], idx_v)

    def gath(j, buf, sem):
        pltpu.async_copy(tab_hbm.at[idx_v.at[pl.ds(j * KG, KG)]], buf, sem)

    def wait_g(buf, sem):
        pltpu.make_async_copy(tab_hbm.at[idx_v.at[pl.ds(0, KG)]], buf, sem).wait()

    def wback(j, buf):
        pltpu.sync_copy(buf, out_hbm.at[pl.ds(base + j * KG, KG)])

    gath(0, buf0, sem0)

    def body(j2, _):
        j = j2 * 2
        wait_g(buf0, sem0)
        gath(j + 1, buf1, sem1)
        wback(j, buf0)
        wait_g(buf1, sem1)

        @pl.when(j + 2 < NBATCH)
        def _():
            gath(j + 2, buf0, sem0)
        wback(j + 1, buf1)
        return 0

    lax.fori_loop(0, NBATCH // 2, body, 0)
    if NBATCH % 2 == 1:
        wait_g(buf0, sem0)
        wback(NBATCH - 1, buf0)


WINDOW = 256
NWIN = (N + WINDOW - 1) // WINDOW
NPAD = NWIN * WINDOW
BEW = 256
NBLK = E // BEW
GTOT = NBLK + NWIN


def _scatter_plan(dst_sorted):
    wstart = jnp.searchsorted(
        dst_sorted, jnp.arange(NWIN + 1, dtype=jnp.int32) * WINDOW).astype(jnp.int32)
    b0 = jnp.minimum(wstart[:-1] // BEW, NBLK - 1)
    b1 = jnp.maximum((wstart[1:] + BEW - 1) // BEW, b0 + 1)
    nsteps = b1 - b0
    csteps = jnp.concatenate([jnp.zeros((1,), jnp.int32),
                              jnp.cumsum(nsteps, dtype=jnp.int32)])
    g = jnp.arange(GTOT, dtype=jnp.int32)
    w = jnp.clip(jnp.searchsorted(csteps, g, side='right').astype(jnp.int32) - 1,
                 0, NWIN - 1)
    j = g - csteps[w]
    pad = g >= csteps[NWIN]
    step_block = jnp.where(pad, 0, b0[w] + j)
    step_win = jnp.where(pad, NWIN - 1, w)
    step_first = jnp.where(pad, 0, (j == 0).astype(jnp.int32))
    step_valid = (~pad).astype(jnp.int32)
    return step_block, step_win, step_first, step_valid


def _tcs_body(sb_ref, sw_ref, sf_ref, sv_ref, msg_ref, dsf_ref, out_ref):
    g = pl.program_id(0)

    @pl.when(sf_ref[g] == 1)
    def _():
        out_ref[...] = jnp.zeros_like(out_ref)

    @pl.when(sv_ref[g] == 1)
    def _():
        base = sw_ref[g] * WINDOW
        dl = dsf_ref[0, 0, :].astype(jnp.int32) - base
        col = lax.broadcasted_iota(jnp.int32, (WINDOW, BEW), 0)
        oh = (col == dl[None, :]).astype(jnp.float32)
        out_ref[...] += jnp.dot(oh, msg_ref[...], preferred_element_type=jnp.float32)


def _tc_scatter(msg, dsf, step_block, step_win, step_first, step_valid):
    grid_spec = pltpu.PrefetchScalarGridSpec(
        num_scalar_prefetch=4,
        grid=(GTOT,),
        in_specs=[
            pl.BlockSpec((BEW, 9 * C), lambda g, sb, sw, sf, sv: (sb[g], 0)),
            pl.BlockSpec((1, 1, BEW), lambda g, sb, sw, sf, sv: (sb[g], 0, 0)),
        ],
        out_specs=pl.BlockSpec((WINDOW, 9 * C), lambda g, sb, sw, sf, sv: (sw[g], 0)),
    )
    return pl.pallas_call(
        _tcs_body,
        grid_spec=grid_spec,
        out_shape=jax.ShapeDtypeStruct((NPAD, 9 * C), jnp.float32),
    )(step_block, step_win, step_first, step_valid, msg, dsf)


def _nodeA_body(h_ref, w1_ref, out_ref):
    out_ref[...] = _pll(h_ref[...], w1_ref[...])


def _nodeA(h2, W1l):
    return pl.pallas_call(
        _nodeA_body,
        grid=(N // BN,),
        in_specs=[
            pl.BlockSpec((BN, 9 * C), lambda i: (i, 0)),
            pl.BlockSpec((3, C, C), lambda i: (0, 0, 0)),
        ],
        out_specs=pl.BlockSpec((BN, 9 * C), lambda i: (i, 0)),
        out_shape=jax.ShapeDtypeStruct((N, 9 * C), jnp.float32),
    )(h2, W1l)


def _nodeB_body(agg_ref, sc_ref, w2_ref, ac_ref, wp_ref, out_ref):
    m = _pll(agg_ref[...], w2_ref[...])
    s = m[:, 0:C]
    ac = ac_ref[...]
    fac = ac[0:1, :] + ac[1:2, :] * s + ac[2:3, :] * (s * s)
    body = jnp.concatenate([m[:, j * C:(j + 1) * C] * fac for j in range(9)], axis=1)
    out_ref[...] = _pll(body, wp_ref[...]) + sc_ref[...]


def _nodeB(agg2, sc2, W2l, acl, Wpl):
    return pl.pallas_call(
        _nodeB_body,
        grid=(N // BN,),
        in_specs=[
            pl.BlockSpec((BN, 9 * C), lambda i: (i, 0)),
            pl.BlockSpec((BN, 9 * C), lambda i: (i, 0)),
            pl.BlockSpec((3, C, C), lambda i: (0, 0, 0)),
            pl.BlockSpec((3, C), lambda i: (0, 0)),
            pl.BlockSpec((3, C, C), lambda i: (0, 0, 0)),
        ],
        out_specs=pl.BlockSpec((BN, 9 * C), lambda i: (i, 0)),
        out_shape=jax.ShapeDtypeStruct((N, 9 * C), jnp.float32),
    )(agg2, sc2, W2l, acl, Wpl)


def kernel(Z, bond_dist, bond_diff, edge_index, embed, W1, Wr1, br1, Wr2, br2,
           Wr3, br3, CGw, W2, acoef, Wp):
    dst_u = edge_index[:, 1]
    perm = jnp.argsort(dst_u)
    ds = dst_u[perm]
    src = edge_index[:, 0][perm]
    bond_dist = bond_dist[perm]
    bond_diff = bond_diff[perm]
    step_block, step_win, step_first, step_valid = _scatter_plan(ds)
    dsf = ds.astype(jnp.float32).reshape(NBLK, 1, BEW)

    r = bond_dist
    edge_dir = bond_diff / (r[:, None] + 1e-8)
    x, y, z = edge_dir[:, 0], edge_dir[:, 1], edge_dir[:, 2]
    s3, s15, s5 = np.sqrt(3.0), np.sqrt(15.0), np.sqrt(5.0)
    Y = jnp.stack([jnp.ones_like(x), s3 * x, s3 * y, s3 * z,
                   s15 * x * y, s15 * y * z, (s5 / 2.0) * (3.0 * z * z - 1.0),
                   s15 * x * z, (s15 / 2.0) * (x * x - y * y)], axis=-1)
    nb = jnp.arange(1, NB + 1, dtype=jnp.float32)
    rr = r[:, None] + 1e-8
    bes = np.sqrt(2.0 / RMAX) * jnp.sin(nb * np.pi * rr / RMAX) / rr
    cut = 0.5 * (jnp.cos(np.pi * r / RMAX) + 1.0) * (r < RMAX).astype(jnp.float32)
    ef = bes * cut[:, None]

    node_scalar = embed[Z]
    h2 = jnp.zeros((N, 9 * C), jnp.float32).at[:, 0:C].set(node_scalar)

    br1_2 = br1.reshape(NL, 1, C)
    br2_2 = br2.reshape(NL, 1, C)
    perm = np.arange(3 * C).reshape(3, C)
    perm = (perm % C) * 3 + (perm // C)
    perm = perm.reshape(-1)
    Wr3_p = Wr3[:, :, perm]
    br3_2 = br3[:, perm].reshape(NL, 1, 3 * C)

    for layer in range(NL):
        cgf = jnp.transpose(CGw[layer], (2, 1, 0)).reshape(9, 81)
        h_up = _nodeA(h2, W1[layer])
        x2 = _get_sc_gather()(h_up, src)
        msg = _edge_stage(ef, Y, x2, Wr1[layer], br1_2[layer], Wr2[layer],
                          br2_2[layer], Wr3_p[layer], br3_2[layer], cgf)
        agg = _tc_scatter(msg, dsf, step_block, step_win, step_first, step_valid)[:N]
        h2 = _nodeB(agg, h2, W2[layer], acoef[layer], Wp[layer])

    return h2.reshape(N, 9, C).transpose(0, 2, 1).reshape(N, 9 * C)
